# baseline (speedup 1.0000x reference)
import functools

import numpy as np
import jax
import jax.numpy as jnp
from jax import lax
from jax.experimental import pallas as pl
from jax.experimental.pallas import tpu as pltpu
from jax.experimental.pallas import tpu_sc as plsc


def _sc_gather(table, idx_grouped, B, D, NC, NS):
    NW, nchunk, ch = idx_grouped.shape
    mesh = plsc.VectorSubcoreMesh(core_axis_name="c", subcore_axis_name="s")

    @functools.partial(
        pl.kernel,
        out_type=jax.ShapeDtypeStruct((B, D), jnp.float32),
        mesh=mesh,
        scratch_types=[
            pltpu.VMEM((nchunk, ch), jnp.int32),
            pltpu.VMEM((ch, D), jnp.float32),
            pltpu.SemaphoreType.DMA,
        ],
    )
    def gather(table_hbm, idx_hbm, out_hbm, idx_v, rows_v, sem):
        wid = lax.axis_index("s") * NC + lax.axis_index("c")
        pltpu.sync_copy(idx_hbm.at[wid], idx_v)
        base = wid * (nchunk * ch)

        def body(k, _):
            pltpu.async_copy(table_hbm.at[idx_v.at[k]], rows_v, sem).wait()
            pltpu.sync_copy(rows_v, out_hbm.at[pl.ds(base + k * ch, ch)])
            return _

        lax.fori_loop(0, nchunk, body, None)

    return gather(table, idx_grouped)


def _fuse(x, emb, G, BB):
    b, c, S = x.shape
    _, N, _ = emb.shape

    def body(x_ref, emb_ref, g_ref, out_ref):
        out_ref[:, :c, :] = x_ref[...]
        for j in range(BB):
            out_ref[j, c:, :] = lax.dot_general(
                emb_ref[j],
                g_ref[...],
                (((0,), (0,)), ((), ())),
                preferred_element_type=jnp.float32,
                precision=lax.Precision.HIGHEST,
            )

    return pl.pallas_call(
        body,
        grid=(b // BB,),
        in_specs=[
            pl.BlockSpec((BB, c, S), lambda i: (i, 0, 0)),
            pl.BlockSpec((BB, N, c), lambda i: (i, 0, 0)),
            pl.BlockSpec((N, S), lambda i: (0, 0)),
        ],
        out_specs=pl.BlockSpec((BB, 2 * c, S), lambda i: (i, 0, 0)),
        out_shape=jax.ShapeDtypeStruct((b, 2 * c, S), jnp.float32),
    )(x, emb, G)


def kernel(x, codes, W_emb):
    b, c, S = x.shape
    _, N = codes.shape
    V, D = W_emb.shape
    B = b * N

    info = plsc.get_sparse_core_info()
    NC, NS = info.num_cores, info.num_subcores
    NW = NC * NS
    per_w = B // NW
    ch = 64
    nchunk = per_w // ch

    idx = codes.reshape(NW, nchunk, ch).astype(jnp.int32)
    emb = _sc_gather(W_emb, idx, B, D, NC, NS)
    emb = emb.reshape(b, N, D)

    src = np.floor(np.arange(S) * (N / S)).astype(np.int32)
    G = jnp.asarray((src[None, :] == np.arange(N)[:, None]).astype(np.float32))

    return _fuse(x, emb, G, BB=8)
